# Initial kernel scaffold; baseline (speedup 1.0000x reference)
import jax
import jax.numpy as jnp
from jax.experimental import pallas as pl


def kernel(x_in, edge_index, batch, W1, b1, W2, b2):
    raise NotImplementedError("write your pallas kernel here")


# speedup vs baseline: 25.6132x; 25.6132x over previous
import functools

import jax
import jax.numpy as jnp
from jax import lax
from jax.experimental import pallas as pl
from jax.experimental.pallas import tpu as pltpu
from jax.experimental.pallas import tpu_sc as plsc

_NC, _NS, _L = 2, 16, 16
_NW = _NC * _NS
_C = 80
_ZR = 64
_HIGH = lax.Precision.HIGHEST


def _sc_mesh():
    return plsc.VectorSubcoreMesh(
        core_axis_name="c", subcore_axis_name="s",
        num_cores=_NC, num_subcores=_NS)


@functools.lru_cache(maxsize=None)
def _make_sc_degree(E, NPAD):
    G = E // (_NW * _C)
    RPT = NPAD // _NS

    def body(dst_hbm, out_hbm, idx_v, ones_v, stage_v, acc_sh):
        cid = lax.axis_index("c")
        sid = lax.axis_index("s")
        wid = cid * _NS + sid

        for j in range(_C // _L):
            ones_v[pl.ds(j * _L, _L)] = jnp.ones((_L,), jnp.float32)

        def zstep(r, carry):
            stage_v[pl.ds(r * _L, _L)] = jnp.zeros((_L,), jnp.float32)
            return carry
        lax.fori_loop(0, RPT // _L, zstep, 0)
        pltpu.sync_copy(stage_v, acc_sh.at[pl.ds(sid * RPT, RPT)])
        plsc.subcore_barrier()

        pltpu.sync_copy(dst_hbm.at[wid], idx_v)

        def step(g, carry):
            pltpu.sync_copy(ones_v, acc_sh.at[idx_v.at[g]], add=True)
            return carry
        lax.fori_loop(0, G, step, 0)

        plsc.subcore_barrier()
        pltpu.sync_copy(acc_sh.at[pl.ds(sid * RPT, RPT)], stage_v)
        pltpu.sync_copy(stage_v, out_hbm.at[pl.ds(cid * NPAD + sid * RPT, RPT)])

    return pl.kernel(
        body,
        out_type=jax.ShapeDtypeStruct((_NC * NPAD,), jnp.float32),
        mesh=_sc_mesh(),
        scratch_types=[
            pltpu.VMEM((G, _C), jnp.int32),
            pltpu.VMEM((_C,), jnp.float32),
            pltpu.VMEM((RPT,), jnp.float32),
            pltpu.VMEM_SHARED((NPAD,), jnp.float32),
        ],
        compiler_params=pltpu.CompilerParams(has_side_effects=True),
    )


@functools.lru_cache(maxsize=None)
def _make_sc_agg(E, NPAD, D, ntab):
    G = E // (_NW * _C)
    assert G % 2 == 1, "pipeline pattern below assumes an odd chunk count"
    RPT = NPAD // _NS

    def body(*refs):
        y_hbms = refs[:ntab]
        src_hbm, dst_hbm, out_hbm = refs[ntab:ntab + 3]
        idxs_v, idxd_v, rows0, rows1, zbuf, acc_sh, sem0, sem1 = refs[ntab + 3:]
        cid = lax.axis_index("c")
        sid = lax.axis_index("s")
        wid = cid * _NS + sid

        def zstep(r, carry):
            for j in range(D // _L):
                zbuf[r, pl.ds(j * _L, _L)] = jnp.zeros((_L,), jnp.float32)
            return carry
        lax.fori_loop(0, _ZR, zstep, 0)

        pltpu.sync_copy(src_hbm.at[wid], idxs_v)
        pltpu.sync_copy(dst_hbm.at[wid], idxd_v)

        for t, y_hbm in enumerate(y_hbms):
            for z in range(RPT // _ZR):
                pltpu.sync_copy(zbuf,
                                acc_sh.at[pl.ds(sid * RPT + z * _ZR, _ZR)])
            plsc.subcore_barrier()

            def gather(g, buf, sem):
                return pltpu.async_copy(y_hbm.at[idxs_v.at[g]], buf, sem)

            def drain_scatter(g, buf, sem):
                pltpu.make_async_copy(y_hbm.at[idxs_v.at[g]], buf, sem).wait()
                pltpu.sync_copy(buf, acc_sh.at[idxd_v.at[g]], add=True)

            gather(0, rows0, sem0)

            def pair(p, carry):
                g0 = 2 * p
                gather(g0 + 1, rows1, sem1)
                drain_scatter(g0, rows0, sem0)
                gather(g0 + 2, rows0, sem0)
                drain_scatter(g0 + 1, rows1, sem1)
                return carry
            lax.fori_loop(0, (G - 1) // 2, pair, 0)
            drain_scatter(G - 1, rows0, sem0)

            plsc.subcore_barrier()
            for z in range(RPT // _ZR):
                r0 = sid * RPT + z * _ZR
                pltpu.sync_copy(acc_sh.at[pl.ds(r0, _ZR)], zbuf)
                pltpu.sync_copy(zbuf, out_hbm.at[t, cid, pl.ds(r0, _ZR)])
            if t + 1 < ntab:
                lax.fori_loop(0, _ZR, zstep, 0)

    return pl.kernel(
        body,
        out_type=jax.ShapeDtypeStruct((ntab, _NC, NPAD, D), jnp.float32),
        mesh=_sc_mesh(),
        scratch_types=[
            pltpu.VMEM((G, _C), jnp.int32),
            pltpu.VMEM((G, _C), jnp.int32),
            pltpu.VMEM((_C, D), jnp.float32),
            pltpu.VMEM((_C, D), jnp.float32),
            pltpu.VMEM((_ZR, D), jnp.float32),
            pltpu.VMEM_SHARED((NPAD, D), jnp.float32),
            pltpu.SemaphoreType.DMA,
            pltpu.SemaphoreType.DMA,
        ],
        compiler_params=pltpu.CompilerParams(use_tc_tiling_on_sc=False,
                                             has_side_effects=True),
    )


def _tc1(x_in, W1, degp3, R):
    N, Fin = x_in.shape
    D = W1.shape[1]
    H = D // 2

    def body(x_ref, w_ref, degp_ref, y1a_ref, y1b_ref, dis_ref):
        deg = degp_ref[0] + degp_ref[1] + 1.0
        dis = lax.rsqrt(deg)
        xw = jnp.dot(x_ref[...], w_ref[...],
                     preferred_element_type=jnp.float32, precision=_HIGH)
        y1 = xw * dis
        y1a_ref[...] = y1[:, :H]
        y1b_ref[...] = y1[:, H:]
        dis_ref[...] = dis

    return pl.pallas_call(
        body,
        grid=(N // R,),
        in_specs=[
            pl.BlockSpec((R, Fin), lambda i: (i, 0)),
            pl.BlockSpec((Fin, D), lambda i: (0, 0)),
            pl.BlockSpec((_NC, R, 1), lambda i: (0, i, 0)),
        ],
        out_specs=[
            pl.BlockSpec((R, H), lambda i: (i, 0)),
            pl.BlockSpec((R, H), lambda i: (i, 0)),
            pl.BlockSpec((R, 1), lambda i: (i, 0)),
        ],
        out_shape=[
            jax.ShapeDtypeStruct((N, H), jnp.float32),
            jax.ShapeDtypeStruct((N, H), jnp.float32),
            jax.ShapeDtypeStruct((N, 1), jnp.float32),
        ],
    )(x_in, W1, degp3)


def _tc2(aggp, y1a, y1b, dis, b1r, W2, R):
    N, H = y1a.shape
    D = 2 * H
    K = W2.shape[1]

    def body(aggpA_ref, aggpB_ref, y1a_ref, y1b_ref, dis_ref, b1_ref, w2_ref,
             x_ref, y2_ref):
        dis = dis_ref[...]
        b1 = b1_ref[...]
        xxA = (aggpA_ref[0, 0] + aggpA_ref[0, 1] + y1a_ref[...]) * dis \
            + b1[:, :H]
        xxB = (aggpB_ref[0, 0] + aggpB_ref[0, 1] + y1b_ref[...]) * dis \
            + b1[:, H:]
        x_ref[...] = jnp.concatenate([xxA, xxB], axis=1)
        w2 = w2_ref[...]
        xw2 = (jnp.dot(xxA, w2[:H], preferred_element_type=jnp.float32,
                       precision=_HIGH)
               + jnp.dot(xxB, w2[H:], preferred_element_type=jnp.float32,
                         precision=_HIGH))
        y2_ref[...] = xw2 * dis

    return pl.pallas_call(
        body,
        grid=(N // R,),
        in_specs=[
            pl.BlockSpec((1, _NC, R, H), lambda i: (0, 0, i, 0)),
            pl.BlockSpec((1, _NC, R, H), lambda i: (1, 0, i, 0)),
            pl.BlockSpec((R, H), lambda i: (i, 0)),
            pl.BlockSpec((R, H), lambda i: (i, 0)),
            pl.BlockSpec((R, 1), lambda i: (i, 0)),
            pl.BlockSpec((1, D), lambda i: (0, 0)),
            pl.BlockSpec((D, K), lambda i: (0, 0)),
        ],
        out_specs=[
            pl.BlockSpec((R, D), lambda i: (i, 0)),
            pl.BlockSpec((R, K), lambda i: (i, 0)),
        ],
        out_shape=[
            jax.ShapeDtypeStruct((N, D), jnp.float32),
            jax.ShapeDtypeStruct((N, K), jnp.float32),
        ],
    )(aggp, aggp, y1a, y1b, dis, b1r, W2)


def _tc3(agg2p, y2, dis, b2r, batchf, x, B, R):
    N, K = y2.shape
    D = x.shape[1]

    def body(agg2p_ref, y2_ref, dis_ref, b2_ref, batch_ref, x_ref, out_ref):
        i = pl.program_id(0)
        logits = ((agg2p_ref[0, 0] + agg2p_ref[0, 1] + y2_ref[...])
                  * dis_ref[...] + b2_ref[...])
        m = jnp.max(logits, axis=1, keepdims=True)
        e = jnp.exp(logits - m)
        S = e / jnp.sum(e, axis=1, keepdims=True)

        @pl.when(i == 0)
        def _():
            out_ref[...] = jnp.zeros_like(out_ref)

        xb = x_ref[...]
        bf = batch_ref[...]
        for b in range(B):
            Sb = jnp.where(bf == float(b), S, 0.0)
            contrib = lax.dot_general(
                Sb, xb, dimension_numbers=(((0,), (0,)), ((), ())),
                preferred_element_type=jnp.float32, precision=_HIGH)
            out_ref[b] = out_ref[b] + contrib

    return pl.pallas_call(
        body,
        grid=(N // R,),
        in_specs=[
            pl.BlockSpec((1, _NC, R, K), lambda i: (0, 0, i, 0)),
            pl.BlockSpec((R, K), lambda i: (i, 0)),
            pl.BlockSpec((R, 1), lambda i: (i, 0)),
            pl.BlockSpec((1, K), lambda i: (0, 0)),
            pl.BlockSpec((R, 1), lambda i: (i, 0)),
            pl.BlockSpec((R, D), lambda i: (i, 0)),
        ],
        out_specs=pl.BlockSpec((B, K, D), lambda i: (0, 0, 0)),
        out_shape=jax.ShapeDtypeStruct((B, K, D), jnp.float32),
    )(agg2p, y2, dis, b2r, batchf, x)


def kernel(x_in, edge_index, batch, W1, b1, W2, b2):
    N = x_in.shape[0]
    E = edge_index.shape[1]
    D = W1.shape[1]
    K = W2.shape[1]
    B = 16
    R = 1000
    NPAD = ((N + _NS * 8 - 1) // (_NS * 8)) * (_NS * 8)
    assert E % (_NW * _C) == 0 and N % R == 0 and D % 2 == 0

    G = E // (_NW * _C)
    src2 = edge_index[0].reshape(_NW, G, _C)
    dst2 = edge_index[1].reshape(_NW, G, _C)

    degp = _make_sc_degree(E, NPAD)(dst2)
    degp3 = degp.reshape(_NC, NPAD, 1)
    y1a, y1b, dis = _tc1(x_in, W1, degp3, R)

    aggp = _make_sc_agg(E, NPAD, D // 2, 2)(y1a, y1b, src2, dst2)
    x, y2 = _tc2(aggp, y1a, y1b, dis, b1.reshape(1, D), W2, R)

    agg2p = _make_sc_agg(E, NPAD, K, 1)(y2, src2, dst2)
    batchf = batch.astype(jnp.float32).reshape(N, 1)
    pooled = _tc3(agg2p, y2, dis, b2.reshape(1, K), batchf, x, B, R)
    return pooled
